# baseline (speedup 1.0000x reference)
import functools

import jax
import jax.numpy as jnp
from jax import lax
from jax.experimental import pallas as pl
from jax.experimental.pallas import tpu as pltpu
from jax.experimental.pallas import tpu_sc as plsc

HIDDEN = 1024
N_EXPERTS = 8
INTER = 512
SHARED_DIM = 512
TILE = 256
CHUNK = 256


def _dotT(a, b):
    return jax.lax.dot_general(a, b, (((1,), (1,)), ((), ())),
                               preferred_element_type=jnp.float32)


def _dot(a, b):
    return jax.lax.dot_general(a, b, (((1,), (0,)), ((), ())),
                               preferred_element_type=jnp.float32)


def _chunked_cumsum(a_bool, T):
    a_bf = a_bool.astype(jnp.bfloat16)
    nch = T // CHUNK
    tri = (jax.lax.broadcasted_iota(jnp.int32, (CHUNK, CHUNK), 0)
           >= jax.lax.broadcasted_iota(jnp.int32, (CHUNK, CHUNK), 1)
           ).astype(jnp.bfloat16)
    incs = []
    sums = []
    for c in range(nch):
        blk = a_bf[c * CHUNK:(c + 1) * CHUNK, :]
        inc = _dot(tri, blk)
        incs.append(inc)
        sums.append(inc[CHUNK - 1:CHUNK, :])
    S = jnp.concatenate(sums, axis=0)
    mlt = (jax.lax.broadcasted_iota(jnp.int32, (nch, nch), 0)
           < jax.lax.broadcasted_iota(jnp.int32, (nch, nch), 1)
           ).astype(jnp.float32)
    O = jax.lax.dot_general(mlt, S, (((0,), (0,)), ((), ())),
                            preferred_element_type=jnp.float32)
    C = jnp.concatenate(
        [incs[c] + O[c:c + 1, :] for c in range(nch)], axis=0)
    totals = O[nch - 1:nch, :] + S[nch - 1:nch, :]
    return C, totals


def _router_body(x_ref, gate_ref, pc_ref, wn1_ref, wn2_ref, te_ref, xbf_ref):
    T = x_ref.shape[0]
    NT = te_ref.shape[0]
    x = x_ref[...]
    xbf_ref[...] = x.astype(jnp.bfloat16)
    logits = _dotT(x, gate_ref[...])
    m = jnp.max(logits, axis=-1, keepdims=True)
    ex = jnp.exp(logits - m)
    p = ex / jnp.sum(ex, axis=-1, keepdims=True)
    idx = jax.lax.broadcasted_iota(jnp.int32, p.shape, 1)
    m1 = jnp.max(p, axis=-1, keepdims=True)
    i1 = jnp.min(jnp.where(p == m1, idx, N_EXPERTS), axis=-1, keepdims=True)
    oh1 = (idx == i1)
    pp2 = jnp.where(oh1, -jnp.inf, p)
    m2 = jnp.max(pp2, axis=-1, keepdims=True)
    i2 = jnp.min(jnp.where(pp2 == m2, idx, N_EXPERTS), axis=-1, keepdims=True)
    oh2 = (idx == i2)
    denom = m1 + m2
    wn1_ref[...] = m1 / denom
    wn2_ref[...] = m2 / denom

    C1, tot1 = _chunked_cumsum(oh1, T)
    C2, tot2 = _chunked_cumsum(oh2, T)
    counts = tot1 + tot2
    padded = jnp.floor((counts + (TILE - 1)) / TILE) * TILE
    mlt8 = (jax.lax.broadcasted_iota(jnp.int32, (N_EXPERTS, N_EXPERTS), 0)
            < jax.lax.broadcasted_iota(jnp.int32, (N_EXPERTS, N_EXPERTS), 1)
            ).astype(jnp.float32)
    pad_off = jax.lax.dot_general(padded, mlt8, (((1,), (0,)), ((), ())),
                                  preferred_element_type=jnp.float32)
    p1f = jnp.sum(jnp.where(oh1, pad_off + C1 - 1.0, 0.0),
                  axis=-1, keepdims=True)
    p2f = jnp.sum(jnp.where(oh2, pad_off + tot1 + C2 - 1.0, 0.0),
                  axis=-1, keepdims=True)
    pc_ref[...] = jnp.concatenate([p1f, p2f], axis=0).astype(jnp.int32)

    starts = (jax.lax.broadcasted_iota(jnp.int32, (NT, 1), 0) * TILE
              ).astype(jnp.float32)
    started = (pad_off <= starts)
    te_ref[...] = (jnp.sum(started.astype(jnp.float32), axis=-1,
                           keepdims=True) - 1.0).astype(jnp.int32)


def _shared_body(xbf_ref, wgu_ref, wd_ref, ysh_ref):
    x = xbf_ref[...]
    gu = _dotT(x, wgu_ref[...].astype(jnp.bfloat16))
    gate = gu[:, :SHARED_DIM]
    up = gu[:, SHARED_DIM:]
    s = gate * jax.nn.sigmoid(gate) * up
    ysh_ref[...] = _dotT(s.astype(jnp.bfloat16),
                         wd_ref[...].astype(jnp.bfloat16))


def _grouped_body(te_ref, xg_ref, w1_ref, w2_ref, ys_ref):
    xt = xg_ref[...]
    h = _dotT(xt, w1_ref[0].astype(jnp.bfloat16))
    h = h * jax.nn.sigmoid(h)
    ys_ref[...] = _dotT(h.astype(jnp.bfloat16),
                        w2_ref[0].astype(jnp.bfloat16)).astype(jnp.bfloat16)


def _combine_body(ysh_ref, yg1_ref, yg2_ref, wn1_ref, wn2_ref, out_ref):
    out_ref[...] = (ysh_ref[...]
                    + wn1_ref[...] * yg1_ref[...].astype(jnp.float32)
                    + wn2_ref[...] * yg2_ref[...].astype(jnp.float32))


def _make_push_dispatch(T, PAD, D, chunk):
    NW = 32
    n_assign = 2 * T
    per_w = n_assign // NW
    assert per_w % chunk == 0
    mesh = plsc.VectorSubcoreMesh(core_axis_name="c", subcore_axis_name="s")

    @functools.partial(
        pl.kernel, mesh=mesh,
        out_type=jax.ShapeDtypeStruct((PAD, D), jnp.int32),
        scratch_types=[
            pltpu.VMEM((chunk,), jnp.int32),
            pltpu.VMEM((chunk, D), jnp.int32),
            pltpu.SemaphoreType.DMA,
        ],
    )
    def push_dispatch(x_hbm, pc_hbm, xg_hbm, idx_v, rows_v, sem):
        wid = lax.axis_index("s") * 2 + lax.axis_index("c")
        for c in range(per_w // chunk):
            aoff = wid * per_w + c * chunk
            toff = jnp.where(aoff < T, aoff, aoff - T)
            pltpu.sync_copy(x_hbm.at[pl.ds(toff, chunk)], rows_v)
            pltpu.sync_copy(pc_hbm.at[pl.ds(aoff, chunk)], idx_v)
            pltpu.async_copy(rows_v, xg_hbm.at[idx_v], sem).wait()

    return push_dispatch


def _make_gather_rows(n_rows, D, dtype, chunk):
    NW = 32
    per_w = n_rows // NW
    assert per_w % chunk == 0 and per_w % 8 == 0
    mesh = plsc.VectorSubcoreMesh(core_axis_name="c", subcore_axis_name="s")

    @functools.partial(
        pl.kernel, mesh=mesh,
        out_type=jax.ShapeDtypeStruct((n_rows, D), dtype),
        scratch_types=[
            pltpu.VMEM((chunk,), jnp.int32),
            pltpu.VMEM((chunk, D), dtype),
            pltpu.SemaphoreType.DMA,
        ],
    )
    def gather_rows(table_hbm, idx_hbm, out_hbm, idx_v, rows_v, sem):
        wid = lax.axis_index("s") * 2 + lax.axis_index("c")
        base = wid * per_w
        for c in range(per_w // chunk):
            off = base + c * chunk
            pltpu.sync_copy(idx_hbm.at[pl.ds(off, chunk)], idx_v)
            pltpu.async_copy(table_hbm.at[idx_v], rows_v, sem).wait()
            pltpu.sync_copy(rows_v, out_hbm.at[pl.ds(off, chunk)])

    return gather_rows


def kernel(hidden_states, gate_w, w1, w2, shared_gate_up_w, shared_down_w):
    orig_shape = hidden_states.shape
    T = orig_shape[0] * orig_shape[1]
    x2d = hidden_states.reshape(T, HIDDEN)
    PAD = 2 * T + N_EXPERTS * TILE
    NT = PAD // TILE

    pc, wn1, wn2, te, xbf = pl.pallas_call(
        _router_body,
        grid=(1,),
        in_specs=[
            pl.BlockSpec((T, HIDDEN), lambda i: (0, 0)),
            pl.BlockSpec((N_EXPERTS, HIDDEN), lambda i: (0, 0)),
        ],
        out_specs=[
            pl.BlockSpec((2 * T, 1), lambda i: (0, 0)),
            pl.BlockSpec((T, 1), lambda i: (0, 0)),
            pl.BlockSpec((T, 1), lambda i: (0, 0)),
            pl.BlockSpec((NT, 1), lambda i: (0, 0)),
            pl.BlockSpec((T, HIDDEN), lambda i: (0, 0)),
        ],
        out_shape=[
            jax.ShapeDtypeStruct((2 * T, 1), jnp.int32),
            jax.ShapeDtypeStruct((T, 1), jnp.float32),
            jax.ShapeDtypeStruct((T, 1), jnp.float32),
            jax.ShapeDtypeStruct((NT, 1), jnp.int32),
            jax.ShapeDtypeStruct((T, HIDDEN), jnp.bfloat16),
        ],
    )(x2d, gate_w)

    pc_flat = pc.reshape(2 * T)

    def _to_i32(a):
        n, d = a.shape
        return jax.lax.bitcast_convert_type(
            a.reshape(n, d // 2, 2), jnp.int32)

    def _to_bf16(a):
        n, d = a.shape
        return jax.lax.bitcast_convert_type(a, jnp.bfloat16).reshape(n, 2 * d)

    xg = _to_bf16(
        _make_push_dispatch(T, PAD, HIDDEN // 2, 64)(_to_i32(xbf), pc_flat))

    ysh = pl.pallas_call(
        _shared_body,
        grid=(1,),
        in_specs=[
            pl.BlockSpec((T, HIDDEN), lambda i: (0, 0)),
            pl.BlockSpec((2 * SHARED_DIM, HIDDEN), lambda i: (0, 0)),
            pl.BlockSpec((HIDDEN, SHARED_DIM), lambda i: (0, 0)),
        ],
        out_specs=pl.BlockSpec((T, HIDDEN), lambda i: (0, 0)),
        out_shape=jax.ShapeDtypeStruct((T, HIDDEN), jnp.float32),
    )(xbf, shared_gate_up_w, shared_down_w)

    ys = pl.pallas_call(
        _grouped_body,
        grid_spec=pltpu.PrefetchScalarGridSpec(
            num_scalar_prefetch=1,
            grid=(NT,),
            in_specs=[
                pl.BlockSpec((TILE, HIDDEN), lambda i, te: (i, 0)),
                pl.BlockSpec((1, INTER, HIDDEN), lambda i, te: (te[i], 0, 0)),
                pl.BlockSpec((1, HIDDEN, INTER), lambda i, te: (te[i], 0, 0)),
            ],
            out_specs=pl.BlockSpec((TILE, HIDDEN), lambda i, te: (i, 0)),
        ),
        out_shape=jax.ShapeDtypeStruct((PAD, HIDDEN), jnp.bfloat16),
    )(te.reshape(NT), xg, w1, w2)

    yg = _to_bf16(
        _make_gather_rows(2 * T, HIDDEN // 2, jnp.int32, 64)(_to_i32(ys),
                                                             pc_flat))

    out = pl.pallas_call(
        _combine_body,
        grid=(1,),
        in_specs=[
            pl.BlockSpec((T, HIDDEN), lambda i: (0, 0)),
            pl.BlockSpec((T, HIDDEN), lambda i: (0, 0)),
            pl.BlockSpec((T, HIDDEN), lambda i: (1, 0)),
            pl.BlockSpec((T, 1), lambda i: (0, 0)),
            pl.BlockSpec((T, 1), lambda i: (0, 0)),
        ],
        out_specs=pl.BlockSpec((T, HIDDEN), lambda i: (0, 0)),
        out_shape=jax.ShapeDtypeStruct((T, HIDDEN), jnp.float32),
    )(ysh, yg, yg, wn1, wn2)

    return out.reshape(orig_shape)


# speedup vs baseline: 7.8630x; 7.8630x over previous
import functools

import jax
import jax.numpy as jnp
from jax.experimental import pallas as pl
from jax.experimental.pallas import tpu as pltpu

HIDDEN = 1024
N_EXPERTS = 8
INTER = 512
SHARED_DIM = 512


def _dotT(a, b):
    return jax.lax.dot_general(a, b, (((1,), (1,)), ((), ())),
                               preferred_element_type=jnp.float32)


def _moe_body(x_ref, gate_ref, w1_ref, w2_ref, wgu_ref, wd_ref,
              out_ref, xbf_ref, comb_ref, h_ref, s_ref):
    e = pl.program_id(0)

    @pl.when(e == 0)
    def _():
        x = x_ref[...]
        xbf_ref[...] = x.astype(jnp.bfloat16)
        logits = _dotT(x, gate_ref[...])
        m = jnp.max(logits, axis=-1, keepdims=True)
        ex = jnp.exp(logits - m)
        p = ex / jnp.sum(ex, axis=-1, keepdims=True)
        idx = jax.lax.broadcasted_iota(jnp.int32, p.shape, 1)
        m1 = jnp.max(p, axis=-1, keepdims=True)
        i1 = jnp.min(jnp.where(p == m1, idx, N_EXPERTS), axis=-1, keepdims=True)
        oh1 = (idx == i1)
        p2 = jnp.where(oh1, -jnp.inf, p)
        m2 = jnp.max(p2, axis=-1, keepdims=True)
        i2 = jnp.min(jnp.where(p2 == m2, idx, N_EXPERTS), axis=-1, keepdims=True)
        oh2 = (idx == i2)
        denom = m1 + m2
        comb_ref[...] = (jnp.where(oh1, m1 / denom, 0.0)
                         + jnp.where(oh2, m2 / denom, 0.0))
        out_ref[...] = jnp.zeros_like(out_ref)

    @pl.when(jnp.logical_and(e >= 1, e <= N_EXPERTS))
    def _():
        y = _dotT(h_ref[...], w2_ref[0].astype(jnp.bfloat16))
        oh_e = (jax.lax.broadcasted_iota(jnp.int32, (1, N_EXPERTS), 1)
                == e - 1)
        w_col = jnp.sum(jnp.where(oh_e, comb_ref[...], 0.0),
                        axis=-1, keepdims=True)
        out_ref[...] += w_col * y

    @pl.when(e < N_EXPERTS)
    def _():
        h = _dotT(xbf_ref[...], w1_ref[0].astype(jnp.bfloat16))
        h_ref[...] = (h * jax.nn.sigmoid(h)).astype(jnp.bfloat16)

    @pl.when(e == N_EXPERTS)
    def _():
        gu = _dotT(xbf_ref[...], wgu_ref[...].astype(jnp.bfloat16))
        gate = gu[:, :SHARED_DIM]
        up = gu[:, SHARED_DIM:]
        s_ref[...] = (gate * jax.nn.sigmoid(gate) * up).astype(jnp.bfloat16)

    @pl.when(e == N_EXPERTS + 1)
    def _():
        out_ref[...] += _dotT(s_ref[...], wd_ref[...].astype(jnp.bfloat16))


def kernel(hidden_states, gate_w, w1, w2, shared_gate_up_w, shared_down_w):
    orig_shape = hidden_states.shape
    T = orig_shape[0] * orig_shape[1]
    x2d = hidden_states.reshape(T, HIDDEN)

    out = pl.pallas_call(
        _moe_body,
        grid=(N_EXPERTS + 2,),
        in_specs=[
            pl.BlockSpec((T, HIDDEN), lambda e: (0, 0)),
            pl.BlockSpec((N_EXPERTS, HIDDEN), lambda e: (0, 0)),
            pl.BlockSpec((1, INTER, HIDDEN),
                         lambda e: (jnp.minimum(e, N_EXPERTS - 1), 0, 0)),
            pl.BlockSpec((1, HIDDEN, INTER),
                         lambda e: (jnp.clip(e - 1, 0, N_EXPERTS - 1), 0, 0)),
            pl.BlockSpec((2 * SHARED_DIM, HIDDEN), lambda e: (0, 0)),
            pl.BlockSpec((HIDDEN, SHARED_DIM), lambda e: (0, 0)),
        ],
        out_specs=pl.BlockSpec((T, HIDDEN), lambda e: (0, 0)),
        out_shape=jax.ShapeDtypeStruct((T, HIDDEN), jnp.float32),
        scratch_shapes=[
            pltpu.VMEM((T, HIDDEN), jnp.bfloat16),
            pltpu.VMEM((T, N_EXPERTS), jnp.float32),
            pltpu.VMEM((T, INTER), jnp.bfloat16),
            pltpu.VMEM((T, SHARED_DIM), jnp.bfloat16),
        ],
    )(x2d, gate_w, w1, w2, shared_gate_up_w, shared_down_w)

    return out.reshape(orig_shape)
